# baseline (speedup 1.0000x reference)
import jax
import jax.numpy as jnp
from jax import lax
from jax.experimental import pallas as pl
from jax.experimental.pallas import tpu as pltpu
from jax.experimental.pallas import tpu_sc as plsc

_INFO = plsc.get_sparse_core_info()
_NC = _INFO.num_cores
_NS = _INFO.num_subcores
_NL = _INFO.num_lanes


def _make(b, s, n_rows, dim):
    groups = _NS // b
    rows_g = dim // groups
    mesh = plsc.VectorSubcoreMesh(core_axis_name="c", subcore_axis_name="s")

    @pl.kernel(
        mesh=mesh,
        compiler_params=pltpu.CompilerParams(use_tc_tiling_on_sc=True,
                                             needs_layout_passes=False),
        out_type=(
            jax.ShapeDtypeStruct((b, dim, s), jnp.float32),
            jax.ShapeDtypeStruct((b, dim, s), jnp.float32),
        ),
        scratch_types=[
            pltpu.VMEM((s,), jnp.int32),
            pltpu.VMEM((rows_g, n_rows), jnp.float32),
            pltpu.VMEM((rows_g, s), jnp.float32),
            pltpu.SemaphoreType.DMA,
            pltpu.SemaphoreType.DMA,
            pltpu.SemaphoreType.DMA,
        ],
    )
    def k(pos_hbm, cos_t_hbm, sin_t_hbm, cos_out, sin_out,
          idx_v, stage_v, out_v, sem_st, sem_ix, sem_o1):
        cid = lax.axis_index("c")
        sid = lax.axis_index("s")
        g = sid % groups
        h = sid // groups
        d0 = g * rows_g

        def pipeline(tab_hbm, out_hbm):
            stage_cpy = pltpu.async_copy(
                tab_hbm.at[pl.ds(d0, rows_g)], stage_v, sem_st)
            idx_cpy = pltpu.async_copy(pos_hbm.at[h], idx_v, sem_ix)
            zeros16 = lax.iota(jnp.int32, _NL) * 0
            row_splats = [zeros16 + r for r in range(rows_g)]
            idx_cpy.wait()
            stage_cpy.wait()

            half = s // (2 * _NL)

            @plsc.parallel_loop(0, half, unroll=16)
            def body1(c):
                ids = idx_v[pl.ds(c * _NL, _NL)]
                for r in range(rows_g):
                    out_v[r, pl.ds(c * _NL, _NL)] = plsc.load_gather(
                        stage_v, [row_splats[r], ids])

            out1 = pltpu.async_copy(
                out_v.at[:, pl.ds(0, s // 2)],
                out_hbm.at[h, pl.ds(d0, rows_g), pl.ds(0, s // 2)], sem_o1)

            @plsc.parallel_loop(half, 2 * half, unroll=16)
            def body2(c):
                ids = idx_v[pl.ds(c * _NL, _NL)]
                for r in range(rows_g):
                    out_v[r, pl.ds(c * _NL, _NL)] = plsc.load_gather(
                        stage_v, [row_splats[r], ids])

            pltpu.sync_copy(
                out_v.at[:, pl.ds(s // 2, s // 2)],
                out_hbm.at[h, pl.ds(d0, rows_g), pl.ds(s // 2, s // 2)])
            out1.wait()

        @pl.when(cid == 0)
        def _():
            pipeline(cos_t_hbm, cos_out)

        @pl.when(cid == 1)
        def _():
            pipeline(sin_t_hbm, sin_out)

    return k


def kernel(qkv, position_ids, cos, sin):
    b, s = position_ids.shape
    n_rows, dim = cos.shape
    cos_ot, sin_ot = _make(b, s, n_rows, dim)(
        position_ids.astype(jnp.int32), cos.T, sin.T)
    return cos_ot.transpose(0, 2, 1), sin_ot.transpose(0, 2, 1)


# speedup vs baseline: 1.1261x; 1.1261x over previous
import jax
import jax.numpy as jnp
from jax import lax
from jax.experimental import pallas as pl
from jax.experimental.pallas import tpu as pltpu
from jax.experimental.pallas import tpu_sc as plsc

_INFO = plsc.get_sparse_core_info()
_NC = _INFO.num_cores
_NS = _INFO.num_subcores
_NL = _INFO.num_lanes


def _make(b, s, n_rows, dim):
    groups = _NS // b
    rows_g = dim // groups
    mesh = plsc.VectorSubcoreMesh(core_axis_name="c", subcore_axis_name="s")

    @pl.kernel(
        mesh=mesh,
        compiler_params=pltpu.CompilerParams(use_tc_tiling_on_sc=True,
                                             needs_layout_passes=False),
        out_type=(
            jax.ShapeDtypeStruct((b, dim, s), jnp.float32),
            jax.ShapeDtypeStruct((b, dim, s), jnp.float32),
        ),
        scratch_types=[
            pltpu.VMEM((s,), jnp.int32),
            pltpu.VMEM((rows_g, n_rows), jnp.float32),
            pltpu.VMEM((rows_g, s), jnp.float32),
            pltpu.SemaphoreType.DMA,
            pltpu.SemaphoreType.DMA,
            pltpu.SemaphoreType.DMA,
        ],
    )
    def k(pos_hbm, cos_t_hbm, sin_t_hbm, cos_out, sin_out,
          idx_v, stage_v, out_v, sem_st, sem_ix, sem_o1):
        cid = lax.axis_index("c")
        sid = lax.axis_index("s")
        g = sid % groups
        h = sid // groups
        d0 = g * rows_g

        def pipeline(tab_hbm, out_hbm):
            stage_cpy = pltpu.async_copy(
                tab_hbm.at[pl.ds(d0, rows_g)], stage_v, sem_st)
            idx_cpy = pltpu.async_copy(pos_hbm.at[h], idx_v, sem_ix)
            zeros16 = lax.iota(jnp.int32, _NL) * 0
            row_splats = [zeros16 + r for r in range(rows_g)]
            idx_cpy.wait()
            stage_cpy.wait()

            half = s // (2 * _NL)

            @plsc.parallel_loop(0, half, unroll=8)
            def body1(c):
                ids = idx_v[pl.ds(c * _NL, _NL)]
                for r in range(rows_g):
                    out_v[r, pl.ds(c * _NL, _NL)] = plsc.load_gather(
                        stage_v, [row_splats[r], ids])

            out1 = pltpu.async_copy(
                out_v.at[:, pl.ds(0, s // 2)],
                out_hbm.at[h, pl.ds(d0, rows_g), pl.ds(0, s // 2)], sem_o1)

            @plsc.parallel_loop(half, 2 * half, unroll=8)
            def body2(c):
                ids = idx_v[pl.ds(c * _NL, _NL)]
                for r in range(rows_g):
                    out_v[r, pl.ds(c * _NL, _NL)] = plsc.load_gather(
                        stage_v, [row_splats[r], ids])

            pltpu.sync_copy(
                out_v.at[:, pl.ds(s // 2, s // 2)],
                out_hbm.at[h, pl.ds(d0, rows_g), pl.ds(s // 2, s // 2)])
            out1.wait()

        @pl.when(cid == 0)
        def _():
            pipeline(cos_t_hbm, cos_out)

        @pl.when(cid == 1)
        def _():
            pipeline(sin_t_hbm, sin_out)

    return k


def kernel(qkv, position_ids, cos, sin):
    b, s = position_ids.shape
    n_rows, dim = cos.shape
    cos_ot, sin_ot = _make(b, s, n_rows, dim)(
        position_ids.astype(jnp.int32), cos.T, sin.T)
    return cos_ot.transpose(0, 2, 1), sin_ot.transpose(0, 2, 1)


# speedup vs baseline: 1.1538x; 1.0246x over previous
import jax
import jax.numpy as jnp
from jax import lax
from jax.experimental import pallas as pl
from jax.experimental.pallas import tpu as pltpu
from jax.experimental.pallas import tpu_sc as plsc

_INFO = plsc.get_sparse_core_info()
_NC = _INFO.num_cores
_NS = _INFO.num_subcores
_NL = _INFO.num_lanes


def _make(b, s, n_rows, dim):
    groups = _NS // b
    rows_g = dim // groups
    mesh = plsc.VectorSubcoreMesh(core_axis_name="c", subcore_axis_name="s")

    @pl.kernel(
        mesh=mesh,
        compiler_params=pltpu.CompilerParams(use_tc_tiling_on_sc=True,
                                             needs_layout_passes=False),
        out_type=(
            jax.ShapeDtypeStruct((b, dim, s), jnp.float32),
            jax.ShapeDtypeStruct((b, dim, s), jnp.float32),
        ),
        scratch_types=[
            pltpu.VMEM((s,), jnp.int32),
            pltpu.VMEM((rows_g, n_rows), jnp.float32),
            pltpu.VMEM((rows_g, s), jnp.float32),
            pltpu.SemaphoreType.DMA,
            pltpu.SemaphoreType.DMA,
            pltpu.SemaphoreType.DMA,
            pltpu.SemaphoreType.DMA,
        ],
    )
    def k(pos_hbm, cos_t_hbm, sin_t_hbm, cos_out, sin_out,
          idx_v, stage_v, out_v, sem_st, sem_s2, sem_ix, sem_o1):
        cid = lax.axis_index("c")
        sid = lax.axis_index("s")
        g = sid % groups
        h = sid // groups
        d0 = g * rows_g

        def pipeline(tab_hbm, out_hbm):
            hg = rows_g // 2
            idx_cpy = pltpu.async_copy(pos_hbm.at[h], idx_v, sem_ix)
            stage0 = pltpu.async_copy(
                tab_hbm.at[pl.ds(d0, hg)], stage_v.at[pl.ds(0, hg)], sem_st)
            stage1 = pltpu.async_copy(
                tab_hbm.at[pl.ds(d0 + hg, hg)], stage_v.at[pl.ds(hg, hg)],
                sem_s2)
            zeros16 = lax.iota(jnp.int32, _NL) * 0
            row_splats = [zeros16 + r for r in range(rows_g)]
            idx_cpy.wait()
            stage0.wait()

            @plsc.parallel_loop(0, s // _NL, unroll=8)
            def body1(c):
                ids = idx_v[pl.ds(c * _NL, _NL)]
                for r in range(hg):
                    out_v[r, pl.ds(c * _NL, _NL)] = plsc.load_gather(
                        stage_v, [row_splats[r], ids])

            out1 = pltpu.async_copy(
                out_v.at[pl.ds(0, hg)],
                out_hbm.at[h, pl.ds(d0, hg)], sem_o1)
            stage1.wait()

            @plsc.parallel_loop(0, s // _NL, unroll=8)
            def body2(c):
                ids = idx_v[pl.ds(c * _NL, _NL)]
                for r in range(hg, rows_g):
                    out_v[r, pl.ds(c * _NL, _NL)] = plsc.load_gather(
                        stage_v, [row_splats[r], ids])

            pltpu.sync_copy(
                out_v.at[pl.ds(hg, hg)],
                out_hbm.at[h, pl.ds(d0 + hg, hg)])
            out1.wait()

        @pl.when(cid == 0)
        def _():
            pipeline(cos_t_hbm, cos_out)

        @pl.when(cid == 1)
        def _():
            pipeline(sin_t_hbm, sin_out)

    return k


def kernel(qkv, position_ids, cos, sin):
    b, s = position_ids.shape
    n_rows, dim = cos.shape
    cos_ot, sin_ot = _make(b, s, n_rows, dim)(
        position_ids.astype(jnp.int32), cos.T, sin.T)
    return cos_ot.transpose(0, 2, 1), sin_ot.transpose(0, 2, 1)
